# Initial kernel scaffold; baseline (speedup 1.0000x reference)
import jax
import jax.numpy as jnp
from jax.experimental import pallas as pl


def kernel(coords, features, cam_position, cam_forward, cam_up):
    raise NotImplementedError("write your pallas kernel here")


# speedup vs baseline: 96.4645x; 96.4645x over previous
import functools
import math

import jax
import jax.numpy as jnp
from jax import lax
from jax.experimental import pallas as pl
from jax.experimental.pallas import tpu as pltpu
from jax.experimental.pallas import tpu_sc as plsc

H = 512
W = 512
FOV = 60.0
ASPECT = 1.0
NEAR = 0.1
FAR = 100.0
ALPHA_THRESHOLD = 0.99
HW = H * W
NPAD = 1 << 18
ROWS = 2048
LANES = 128
LOG2N = 18
LOG_EPS = math.log(1.0 - ALPHA_THRESHOLD)
A_MAX = 0.99999994
NW = 32
PIX_PER_W = HW // NW
CHUNK = 2048
BIG = HW


def _tc_body(karr_ref, jarr_ref, pix_ref_in, nd_ref, a_ref, r_ref, g_ref,
             b_ref, pix_ref, cr_ref, cg_ref, cb_ref, bnd_ref):
    row = lax.broadcasted_iota(jnp.int32, (ROWS, LANES), 0)
    lane = lax.broadcasted_iota(jnp.int32, (ROWS, LANES), 1)
    flat = row * LANES + lane

    pix = pix_ref_in[...]
    nd = nd_ref[...]
    av = jnp.minimum(a_ref[...], A_MAX)
    rv = r_ref[...]
    gv = g_ref[...]
    bv = b_ref[...]
    idx = flat

    karr = karr_ref[...]
    jarr = jarr_ref[...]
    npass = LOG2N * (LOG2N + 1) // 2
    tiny = lax.broadcasted_iota(jnp.int32, (2, 128), 0) * 128 + \
        lax.broadcasted_iota(jnp.int32, (2, 128), 1)

    def sort_pass(t, arrs):
        pixs, nds, idxs = arrs[0], arrs[1], arrs[2]
        kv = jnp.sum(jnp.where(tiny == t, karr, 0))
        jv = jnp.sum(jnp.where(tiny == t, jarr, 0))
        desc = ((flat >> kv) & 1) == 1
        self_hi = ((flat >> jv) & 1) == 1
        s_lane = jnp.int32(1) << jv
        s_row = jnp.int32(1) << jnp.maximum(jv - 7, 0)

        def lane_rolls(_):
            neg = jnp.int32(LANES) - s_lane
            return tuple(pltpu.roll(a, neg, axis=1) for a in arrs) + \
                tuple(pltpu.roll(a, s_lane, axis=1) for a in arrs)

        def row_rolls(_):
            neg = jnp.int32(ROWS) - s_row
            return tuple(pltpu.roll(a, neg, axis=0) for a in arrs) + \
                tuple(pltpu.roll(a, s_row, axis=0) for a in arrs)

        rolled = lax.cond(jv < 7, lane_rolls, row_rolls, 0)
        na = len(arrs)
        part = tuple(jnp.where(self_hi, rolled[na + i], rolled[i])
                     for i in range(na))
        p_lo = jnp.where(self_hi, part[0], pixs)
        p_hi = jnp.where(self_hi, pixs, part[0])
        d_lo = jnp.where(self_hi, part[1], nds)
        d_hi = jnp.where(self_hi, nds, part[1])
        i_lo = jnp.where(self_hi, part[2], idxs)
        i_hi = jnp.where(self_hi, idxs, part[2])
        cmp = ((p_lo < p_hi)
               | ((p_lo == p_hi)
                  & ((d_lo < d_hi) | ((d_lo == d_hi) & (i_lo < i_hi)))))
        swap = cmp == desc
        return tuple(jnp.where(swap, part[i], a)
                     for i, a in enumerate(arrs))

    pix, nd, _, av, rv, gv, bv = lax.fori_loop(
        0, npass, sort_pass, (pix, nd, idx, av, rv, gv, bv))

    bix = lax.broadcasted_iota(jnp.int32, (8, LANES), 0) * LANES + \
        lax.broadcasted_iota(jnp.int32, (8, LANES), 1)

    def bnd_step(t, acc):
        cnt = jnp.sum((pix < t * PIX_PER_W).astype(jnp.int32))
        return acc + jnp.where(bix == t, cnt, 0)

    bnd_ref[...] = lax.fori_loop(0, NW + 1, bnd_step,
                                 jnp.zeros((8, LANES), jnp.int32))

    def shift_up(t, s, fill):
        def lane_case(_):
            u = pltpu.roll(t, s, axis=1)
            return jnp.where(lane >= s, u, pltpu.roll(u, 1, axis=0))

        def row_case(_):
            return pltpu.roll(t, s >> 7, axis=0)

        u = lax.cond(s < LANES, lane_case, row_case, 0)
        return jnp.where(flat >= s, u, fill)

    prev_pix = jnp.where(lane >= 1, pltpu.roll(pix, 1, axis=1),
                         pltpu.roll(pltpu.roll(pix, 1, axis=1), 1, axis=0))
    prev_pix = jnp.where(flat >= 1, prev_pix, -1)
    first = (pix != prev_pix).astype(jnp.int32)
    nxt_pix = jnp.where(
        lane < LANES - 1, pltpu.roll(pix, LANES - 1, axis=1),
        pltpu.roll(pltpu.roll(pix, LANES - 1, axis=1), ROWS - 1, axis=0))
    nxt_pix = jnp.where(flat < NPAD - 1, nxt_pix, -2)
    last = pix != nxt_pix

    la = jnp.log1p(-av)

    def scan_a(st, carry):
        lv, f = carry
        s = jnp.int32(1) << st
        pv = shift_up(lv, s, 0.0)
        pf = shift_up(f, s, 1)
        lv = jnp.where(f > 0, lv, lv + pv)
        return lv, f | pf

    lv, _ = lax.fori_loop(0, LOG2N, scan_a, (la, first))

    l_excl = lv - la
    incl = l_excl > LOG_EPS
    wgt = jnp.where(incl, av * jnp.exp(-lv), 0.0)
    tr = rv * wgt
    tg = gv * wgt
    tb = bv * wgt
    lm = jnp.where(incl, lv, jnp.float32(1e30))

    def scan_b(st, carry):
        lmv, trv, tgv, tbv, f = carry
        s = jnp.int32(1) << st
        plm = shift_up(lmv, s, jnp.float32(1e30))
        ptr = shift_up(trv, s, 0.0)
        ptg = shift_up(tgv, s, 0.0)
        ptb = shift_up(tbv, s, 0.0)
        pf = shift_up(f, s, 1)
        keep = f > 0
        lmv = jnp.where(keep, lmv, jnp.minimum(lmv, plm))
        trv = jnp.where(keep, trv, trv + ptr)
        tgv = jnp.where(keep, tgv, tgv + ptg)
        tbv = jnp.where(keep, tbv, tbv + ptb)
        return lmv, trv, tgv, tbv, f | pf

    lm, tr, tg, tb, _ = lax.fori_loop(0, LOG2N, scan_b,
                                      (lm, tr, tg, tb, first))

    tm = jnp.exp(lm)
    alpha = 1.0 - tm
    pos = alpha > 0.0
    denom = jnp.where(pos, alpha, 1.0)
    cr_ref[...] = jnp.where(pos, tm * tr / denom, 0.0)
    cg_ref[...] = jnp.where(pos, tm * tg / denom, 0.0)
    cb_ref[...] = jnp.where(pos, tm * tb / denom, 0.0)
    pix_ref[...] = jnp.where(last, pix, BIG)


def _pass_schedule():
    ks = []
    js = []
    for k in range(1, LOG2N + 1):
        for j in range(k - 1, -1, -1):
            ks.append(k)
            js.append(j)
    pad = 256 - len(ks)
    karr = jnp.asarray(ks + [0] * pad, jnp.int32).reshape(2, 128)
    jarr = jnp.asarray(js + [0] * pad, jnp.int32).reshape(2, 128)
    return karr, jarr


def _tc_call(pix2, nd2, a2, r2, g2, b2):
    karr, jarr = _pass_schedule()
    return pl.pallas_call(
        _tc_body,
        in_specs=[pl.BlockSpec()] * 8,
        out_shape=[
            jax.ShapeDtypeStruct((ROWS, LANES), jnp.int32),
            jax.ShapeDtypeStruct((ROWS, LANES), jnp.float32),
            jax.ShapeDtypeStruct((ROWS, LANES), jnp.float32),
            jax.ShapeDtypeStruct((ROWS, LANES), jnp.float32),
            jax.ShapeDtypeStruct((8, LANES), jnp.int32),
        ],
        compiler_params=pltpu.CompilerParams(
            vmem_limit_bytes=100 * 1024 * 1024),
    )(karr, jarr, pix2, nd2, a2, r2, g2, b2)


def _make_sc_scatter():
    mesh = plsc.VectorSubcoreMesh(core_axis_name="c", subcore_axis_name="s",
                                  num_cores=2, num_subcores=16)

    @functools.partial(
        pl.kernel,
        out_type=[jax.ShapeDtypeStruct((HW,), jnp.float32)] * 3,
        mesh=mesh,
        scratch_types=[
            pltpu.VMEM((48,), jnp.int32),
            pltpu.VMEM((CHUNK,), jnp.int32),
            pltpu.VMEM((CHUNK,), jnp.float32),
            pltpu.VMEM((CHUNK,), jnp.float32),
            pltpu.VMEM((CHUNK,), jnp.float32),
            pltpu.VMEM((PIX_PER_W + 16,), jnp.float32),
            pltpu.VMEM((PIX_PER_W + 16,), jnp.float32),
            pltpu.VMEM((PIX_PER_W + 16,), jnp.float32),
        ],
        compiler_params=pltpu.CompilerParams(needs_layout_passes=False),
    )
    def sc_scatter(pix_hbm, r_hbm, g_hbm, b_hbm, bnd_hbm, outr, outg, outb,
                   bnd_v, pix_v, rv, gv, bv, imr, img, imb):
        wid = lax.axis_index("s") * 2 + lax.axis_index("c")
        pltpu.sync_copy(bnd_hbm, bnd_v)
        start = bnd_v[pl.ds(wid, 16)][0]
        end = bnd_v[pl.ds(wid + 1, 16)][0]
        base = wid * PIX_PER_W
        zeros16 = jnp.zeros((16,), jnp.float32)

        def zbody(i, c):
            imr[pl.ds(i * 16, 16)] = zeros16
            img[pl.ds(i * 16, 16)] = zeros16
            imb[pl.ds(i * 16, 16)] = zeros16
            return c

        lax.fori_loop(0, PIX_PER_W // 16 + 1, zbody, 0)
        l16 = lax.iota(jnp.int32, 16)

        start_al = (start // 8) * 8
        nch = lax.max(jnp.int32(0), (end - start_al + CHUNK - 1) // CHUNK)

        def cbody(k, c):
            cs = pl.multiple_of(start_al + k * CHUNK, 8)
            pltpu.sync_copy(pix_hbm.at[pl.ds(cs, CHUNK)], pix_v)
            pltpu.sync_copy(r_hbm.at[pl.ds(cs, CHUNK)], rv)
            pltpu.sync_copy(g_hbm.at[pl.ds(cs, CHUNK)], gv)
            pltpu.sync_copy(b_hbm.at[pl.ds(cs, CHUNK)], bv)

            def ibody(i, c2):
                pv = pix_v[pl.ds(i * 16, 16)]
                m = (pv >= base) & (pv < base + PIX_PER_W)
                loc = jnp.where(m, pv - base, PIX_PER_W + l16)
                plsc.store_scatter(imr, [loc], rv[pl.ds(i * 16, 16)])
                plsc.store_scatter(img, [loc], gv[pl.ds(i * 16, 16)])
                plsc.store_scatter(imb, [loc], bv[pl.ds(i * 16, 16)])
                return c2

            lax.fori_loop(0, CHUNK // 16, ibody, 0)
            return c

        lax.fori_loop(0, nch, cbody, 0)
        pltpu.sync_copy(imr.at[pl.ds(0, PIX_PER_W)],
                        outr.at[pl.ds(base, PIX_PER_W)])
        pltpu.sync_copy(img.at[pl.ds(0, PIX_PER_W)],
                        outg.at[pl.ds(base, PIX_PER_W)])
        pltpu.sync_copy(imb.at[pl.ds(0, PIX_PER_W)],
                        outb.at[pl.ds(base, PIX_PER_W)])

    return sc_scatter


_sc_scatter_cache = []


def _get_sc_scatter():
    if not _sc_scatter_cache:
        _sc_scatter_cache.append(_make_sc_scatter())
    return _sc_scatter_cache[0]


def _proj_matrix():
    fov_rad = FOV * math.pi / 180.0
    fc = 1.0 / math.tan(fov_rad / 2.0)
    proj = jnp.zeros((4, 4), jnp.float32)
    proj = proj.at[0, 0].set(fc / ASPECT)
    proj = proj.at[1, 1].set(fc)
    proj = proj.at[2, 2].set((FAR + NEAR) / (NEAR - FAR))
    proj = proj.at[2, 3].set(2.0 * FAR * NEAR / (NEAR - FAR))
    proj = proj.at[3, 2].set(-1.0)
    return proj


@jax.jit
def kernel(coords, features, cam_position, cam_forward, cam_up):
    f = cam_forward / jnp.linalg.norm(cam_forward)
    r = jnp.cross(f, cam_up)
    r = r / jnp.linalg.norm(r)
    u = jnp.cross(r, f)
    view = jnp.eye(4, dtype=jnp.float32)
    view = view.at[0, :3].set(r).at[1, :3].set(u).at[2, :3].set(-f)
    t = -jnp.stack([jnp.dot(r, cam_position), jnp.dot(u, cam_position),
                    -jnp.dot(f, cam_position)])
    view = view.at[:3, 3].set(t)
    mvp = _proj_matrix() @ view

    n = coords.shape[0]
    coords_h = jnp.concatenate([coords, jnp.ones((n, 1), coords.dtype)], -1)
    clip = coords_h @ mvp.T
    ndc = clip[:, :3] / (clip[:, 3:4] + 1e-08)
    visible = ((ndc[:, 0] >= -1) & (ndc[:, 0] <= 1)
               & (ndc[:, 1] >= -1) & (ndc[:, 1] <= 1)
               & (ndc[:, 2] >= -1) & (ndc[:, 2] <= 1))
    ndc_s = jnp.nan_to_num(ndc, nan=0.0, posinf=0.0, neginf=0.0)
    sx = jnp.clip(((ndc_s[:, 0] + 1.0) / 2.0 * W).astype(jnp.int32), 0, W - 1)
    sy = jnp.clip(((1.0 - ndc_s[:, 1]) / 2.0 * H).astype(jnp.int32), 0, H - 1)
    pix_in = jnp.where(visible, sy * W + sx, BIG)
    nd_in = jnp.where(visible, -ndc_s[:, 2], 0.0)
    a_in = jnp.where(visible, features[:, 5], 0.0)

    pad = NPAD - n
    pix2 = jnp.pad(pix_in, (0, pad), constant_values=BIG).reshape(ROWS, LANES)
    nd2 = jnp.pad(nd_in, (0, pad)).reshape(ROWS, LANES)
    a2 = jnp.pad(a_in, (0, pad)).reshape(ROWS, LANES)
    r2 = jnp.pad(features[:, 0], (0, pad)).reshape(ROWS, LANES)
    g2 = jnp.pad(features[:, 1], (0, pad)).reshape(ROWS, LANES)
    b2 = jnp.pad(features[:, 2], (0, pad)).reshape(ROWS, LANES)

    pix, cr, cg, cb, bnd = _tc_call(pix2, nd2, a2, r2, g2, b2)

    pixf = jnp.concatenate([pix.reshape(-1),
                            jnp.full((CHUNK,), BIG, jnp.int32)])
    crf = jnp.concatenate([cr.reshape(-1), jnp.zeros((CHUNK,), jnp.float32)])
    cgf = jnp.concatenate([cg.reshape(-1), jnp.zeros((CHUNK,), jnp.float32)])
    cbf = jnp.concatenate([cb.reshape(-1), jnp.zeros((CHUNK,), jnp.float32)])
    bndf = jnp.concatenate([bnd.reshape(-1)[:NW + 1],
                            jnp.zeros((48 - (NW + 1),), jnp.int32)])

    outr, outg, outb = _get_sc_scatter()(pixf, crf, cgf, cbf, bndf)
    return jnp.stack([outr, outg, outb]).reshape(3, H, W)
